# baseline (speedup 1.0000x reference)
import functools

import jax
import jax.numpy as jnp
from jax import lax
from jax.experimental import pallas as pl
from jax.experimental.pallas import tpu as pltpu
from jax.experimental.pallas import tpu_sc as plsc

N = 10000
E = 320000
D = 128
LAYERS = 4
BN_EPS = 1e-5

NC = 2
NS = 16
NT = NC * NS
K = 80
PER_TILE = E // NT
NBUF = 3
CHUNKS = 126
GROUPS = CHUNKS // NBUF
UNROLL = 3
AGG_ROWS = N + 8
ROWS_PER_TILE = 624
ROWS_TAIL = N - NS * ROWS_PER_TILE

_mesh = plsc.VectorSubcoreMesh(core_axis_name="c", subcore_axis_name="s")


@functools.partial(
    pl.kernel,
    mesh=_mesh,
    out_type=jax.ShapeDtypeStruct((NC, N, D), jnp.float32),
    scratch_types=[
        pltpu.VMEM((UNROLL, NBUF, 2, K), jnp.int32),
        pltpu.VMEM((NBUF, K, D), jnp.float32),
        pltpu.VMEM_SHARED((AGG_ROWS, D), jnp.float32),
    ]
    + [pltpu.SemaphoreType.DMA] * (3 * NBUF),
)
def _sc_agg(h_hbm, idx_hbm, zeros_hbm, out_hbm,
            idx_v, rows_v, agg_sh, *sems):
    c = lax.axis_index("c")
    s = lax.axis_index("s")
    sem_i = sems[:NBUF]
    sem_g = sems[NBUF:2 * NBUF]
    sem_s = sems[2 * NBUF:]
    tid = c * NS + s

    r0 = s * ROWS_PER_TILE
    pltpu.sync_copy(zeros_hbm.at[pl.ds(0, ROWS_PER_TILE)],
                    agg_sh.at[pl.ds(r0, ROWS_PER_TILE)])

    @pl.when(s == NS - 1)
    def _zero_tail():
        rt = NS * ROWS_PER_TILE
        pltpu.sync_copy(zeros_hbm.at[pl.ds(0, ROWS_TAIL)],
                        agg_sh.at[pl.ds(rt, ROWS_TAIL)])

    plsc.subcore_barrier()

    def _idx_load(group, slot):
        return pltpu.make_async_copy(idx_hbm.at[tid, group], idx_v.at[slot],
                                     sem_i[slot])

    def _gather(slot, row, b):
        return pltpu.make_async_copy(h_hbm.at[idx_v.at[slot, row, 0]],
                                     rows_v.at[b], sem_g[b])

    def _scatter(slot, row, b):
        return pltpu.make_async_copy(rows_v.at[b],
                                     agg_sh.at[idx_v.at[slot, row, 1]],
                                     sem_s[b])

    _idx_load(0, 0).start()
    _idx_load(1, 1).start()
    _idx_load(0, 0).wait()
    for b in range(NBUF):
        _gather(0, b, b).start()

    last = GROUPS // UNROLL - 1

    def body(u, carry):
        for p in range(UNROLL):
            slot = p
            for b in range(NBUF):
                _gather(slot, b, b).wait()
                _scatter(slot, b, b).start(add=True)
            for b in range(NBUF):
                _scatter(slot, b, b).wait()
            g2 = UNROLL * u + p + 2
            slot2 = (p + 2) % UNROLL
            if p == 0:
                _idx_load(g2, slot2).start()
            else:
                @pl.when(u < last)
                def _():
                    _idx_load(g2, slot2).start()
            slotn = (p + 1) % UNROLL
            if p < UNROLL - 1:
                _idx_load(0, slotn).wait()
                for b in range(NBUF):
                    _gather(slotn, b, b).start()
            else:
                @pl.when(u < last)
                def _():
                    _idx_load(0, slotn).wait()
                    for b in range(NBUF):
                        _gather(slotn, b, b).start()
        return carry

    lax.fori_loop(0, GROUPS // UNROLL, body, 0)

    plsc.subcore_barrier()
    pltpu.sync_copy(agg_sh.at[pl.ds(r0, ROWS_PER_TILE)],
                    out_hbm.at[c, pl.ds(r0, ROWS_PER_TILE)])

    @pl.when(s == NS - 1)
    def _tail():
        rt = NS * ROWS_PER_TILE
        pltpu.sync_copy(agg_sh.at[pl.ds(rt, ROWS_TAIL)],
                        out_hbm.at[c, pl.ds(rt, ROWS_TAIL)])


BLK = 1000


def _mlp_body(h_ref, a0_ref, a1_ref, w1_ref, s1_ref, t1_ref,
              w2_ref, s2_ref, t2_ref, o_ref, *, final_relu):
    z = h_ref[...] + a0_ref[...] + a1_ref[...]
    z = jnp.dot(z, w1_ref[...], preferred_element_type=jnp.float32)
    z = z * s1_ref[...] + t1_ref[...]
    z = jnp.maximum(z, 0.0)
    z = jnp.dot(z, w2_ref[...], preferred_element_type=jnp.float32)
    z = z * s2_ref[...] + t2_ref[...]
    if final_relu:
        z = jnp.maximum(z, 0.0)
    o_ref[...] = z


def _mlp(h, a0, a1, w1, s1, t1, w2, s2, t2, final_relu):
    row = lambda i: (i, 0)
    fixed = lambda i: (0, 0)
    return pl.pallas_call(
        functools.partial(_mlp_body, final_relu=final_relu),
        grid=(N // BLK,),
        in_specs=[
            pl.BlockSpec((BLK, D), row),
            pl.BlockSpec((BLK, D), row),
            pl.BlockSpec((BLK, D), row),
            pl.BlockSpec((D, D), fixed),
            pl.BlockSpec((1, D), fixed),
            pl.BlockSpec((1, D), fixed),
            pl.BlockSpec((D, D), fixed),
            pl.BlockSpec((1, D), fixed),
            pl.BlockSpec((1, D), fixed),
        ],
        out_specs=pl.BlockSpec((BLK, D), row),
        out_shape=jax.ShapeDtypeStruct((N, D), jnp.float32),
    )(h, a0, a1, w1, s1, t1, w2, s2, t2)


def kernel(x, edge_index, w1, b1, g1, be1, rm1, rv1, w2, b2, g2, be2, rm2, rv2):
    pad = CHUNKS * K - PER_TILE
    src = edge_index[0].astype(jnp.int32).reshape(NT, PER_TILE)
    dst = edge_index[1].astype(jnp.int32).reshape(NT, PER_TILE)
    src = jnp.concatenate(
        [src, jnp.zeros((NT, pad), jnp.int32)], axis=1)
    dst = jnp.concatenate(
        [dst, jnp.full((NT, pad), N, jnp.int32)], axis=1)
    src = src.reshape(NT, GROUPS, NBUF, K)
    dst = dst.reshape(NT, GROUPS, NBUF, K)
    idx = jnp.stack([src, dst], axis=3)
    s1 = g1 * lax.rsqrt(rv1 + BN_EPS)
    t1 = (b1 - rm1) * s1 + be1
    s2 = g2 * lax.rsqrt(rv2 + BN_EPS)
    t2 = (b2 - rm2) * s2 + be2
    zeros = jnp.zeros((ROWS_PER_TILE, D), jnp.float32)
    h = x.astype(jnp.float32)
    for l in range(LAYERS):
        parts = _sc_agg(h, idx, zeros)
        h = _mlp(h, parts[0], parts[1], w1[l],
                 s1[l][None, :], t1[l][None, :],
                 w2[l], s2[l][None, :], t2[l][None, :],
                 l < LAYERS - 1)
    return h


# speedup vs baseline: 1.4937x; 1.4937x over previous
import functools

import jax
import jax.numpy as jnp
from jax import lax
from jax.experimental import pallas as pl
from jax.experimental.pallas import tpu as pltpu
from jax.experimental.pallas import tpu_sc as plsc

N = 10000
E = 320000
D = 128
LAYERS = 4
BN_EPS = 1e-5

NC = 2
NS = 16
NT = NC * NS
K = 40
PER_TILE = E // NT
NBUF = 3
SETS = 3
CHUNKS = 252
GROUPS = CHUNKS // NBUF
GROUPS_ALLOC = GROUPS + 2
AGG_ROWS = N + 8
ROWS_PER_TILE = 624
ROWS_TAIL = N - NS * ROWS_PER_TILE

_mesh = plsc.VectorSubcoreMesh(core_axis_name="c", subcore_axis_name="s")


@functools.partial(
    pl.kernel,
    mesh=_mesh,
    out_type=jax.ShapeDtypeStruct((NC, N, D), jnp.float32),
    scratch_types=[
        pltpu.VMEM((SETS, NBUF, 2, K), jnp.int32),
        pltpu.VMEM((SETS, NBUF, K, D), jnp.float32),
        pltpu.VMEM_SHARED((AGG_ROWS, D), jnp.float32),
    ]
    + [pltpu.SemaphoreType.DMA] * (3 * SETS),
)
def _sc_agg(h_hbm, idx_hbm, zeros_hbm, out_hbm,
            idx_v, rows_v, agg_sh, *sems):
    c = lax.axis_index("c")
    s = lax.axis_index("s")
    sem_i = sems[:SETS]
    sem_g = sems[SETS:2 * SETS]
    sem_s = sems[2 * SETS:]
    tid = c * NS + s

    r0 = s * ROWS_PER_TILE
    pltpu.sync_copy(zeros_hbm.at[pl.ds(0, ROWS_PER_TILE)],
                    agg_sh.at[pl.ds(r0, ROWS_PER_TILE)])

    @pl.when(s == NS - 1)
    def _zero_tail():
        rt = NS * ROWS_PER_TILE
        pltpu.sync_copy(zeros_hbm.at[pl.ds(0, ROWS_TAIL)],
                        agg_sh.at[pl.ds(rt, ROWS_TAIL)])

    plsc.subcore_barrier()

    def _idx_load(group, slot):
        return pltpu.make_async_copy(idx_hbm.at[tid, group], idx_v.at[slot],
                                     sem_i[slot])

    def _gather(m, b):
        return pltpu.make_async_copy(h_hbm.at[idx_v.at[m, b, 0]],
                                     rows_v.at[m, b], sem_g[m])

    def _scatter(m, b):
        return pltpu.make_async_copy(rows_v.at[m, b],
                                     agg_sh.at[idx_v.at[m, b, 1]],
                                     sem_s[m])

    def _group(g, m, nxt=True, wait_prev=True, load=True):
        for b in range(NBUF):
            _gather(m, b).wait()
        for b in range(NBUF):
            _scatter(m, b).start(add=True)
        if nxt:
            mn = (m + 1) % SETS
            _idx_load(0, mn).wait()
            for b in range(NBUF):
                _gather(mn, b).start()
        if wait_prev:
            mp = (m + 2) % SETS
            for b in range(NBUF):
                _scatter(mp, b).wait()
        if load:
            _idx_load(g + 2, (m + 2) % SETS).start()

    _idx_load(0, 0).start()
    _idx_load(1, 1).start()
    _idx_load(0, 0).wait()
    for b in range(NBUF):
        _gather(0, b).start()
    _group(0, 0, wait_prev=False)
    _group(1, 1)

    def body(u, carry):
        g = SETS * u + 2
        _group(g, 2)
        _group(g + 1, 0)
        _group(g + 2, 1)
        return carry

    lax.fori_loop(0, (GROUPS - 3) // SETS, body, 0)

    _group(GROUPS - 1, 2, nxt=False, load=False)
    for b in range(NBUF):
        _scatter(2, b).wait()
    _idx_load(0, 0).wait()

    plsc.subcore_barrier()
    pltpu.sync_copy(agg_sh.at[pl.ds(r0, ROWS_PER_TILE)],
                    out_hbm.at[c, pl.ds(r0, ROWS_PER_TILE)])

    @pl.when(s == NS - 1)
    def _tail():
        rt = NS * ROWS_PER_TILE
        pltpu.sync_copy(agg_sh.at[pl.ds(rt, ROWS_TAIL)],
                        out_hbm.at[c, pl.ds(rt, ROWS_TAIL)])


BLK = 1000


def _mlp_body(h_ref, a0_ref, a1_ref, w1_ref, s1_ref, t1_ref,
              w2_ref, s2_ref, t2_ref, o_ref, *, final_relu):
    z = h_ref[...] + a0_ref[...] + a1_ref[...]
    z = jnp.dot(z, w1_ref[...], preferred_element_type=jnp.float32)
    z = z * s1_ref[...] + t1_ref[...]
    z = jnp.maximum(z, 0.0)
    z = jnp.dot(z, w2_ref[...], preferred_element_type=jnp.float32)
    z = z * s2_ref[...] + t2_ref[...]
    if final_relu:
        z = jnp.maximum(z, 0.0)
    o_ref[...] = z


def _mlp(h, a0, a1, w1, s1, t1, w2, s2, t2, final_relu):
    row = lambda i: (i, 0)
    fixed = lambda i: (0, 0)
    return pl.pallas_call(
        functools.partial(_mlp_body, final_relu=final_relu),
        grid=(N // BLK,),
        in_specs=[
            pl.BlockSpec((BLK, D), row),
            pl.BlockSpec((BLK, D), row),
            pl.BlockSpec((BLK, D), row),
            pl.BlockSpec((D, D), fixed),
            pl.BlockSpec((1, D), fixed),
            pl.BlockSpec((1, D), fixed),
            pl.BlockSpec((D, D), fixed),
            pl.BlockSpec((1, D), fixed),
            pl.BlockSpec((1, D), fixed),
        ],
        out_specs=pl.BlockSpec((BLK, D), row),
        out_shape=jax.ShapeDtypeStruct((N, D), jnp.float32),
    )(h, a0, a1, w1, s1, t1, w2, s2, t2)


def kernel(x, edge_index, w1, b1, g1, be1, rm1, rv1, w2, b2, g2, be2, rm2, rv2):
    pad = CHUNKS * K - PER_TILE
    src = edge_index[0].astype(jnp.int32).reshape(NT, PER_TILE)
    dst = edge_index[1].astype(jnp.int32).reshape(NT, PER_TILE)
    pad_idx = jnp.broadcast_to(
        jnp.arange(pad, dtype=jnp.int32) % 8, (NT, pad))
    src = jnp.concatenate([src, pad_idx], axis=1)
    dst = jnp.concatenate([dst, N + pad_idx], axis=1)
    src = src.reshape(NT, GROUPS, NBUF, K)
    dst = dst.reshape(NT, GROUPS, NBUF, K)
    idx = jnp.stack([src, dst], axis=3)
    idx = jnp.concatenate(
        [idx, jnp.zeros((NT, GROUPS_ALLOC - GROUPS, NBUF, 2, K), jnp.int32)],
        axis=1)
    s1 = g1 * lax.rsqrt(rv1 + BN_EPS)
    t1 = (b1 - rm1) * s1 + be1
    s2 = g2 * lax.rsqrt(rv2 + BN_EPS)
    t2 = (b2 - rm2) * s2 + be2
    zeros = jnp.zeros((ROWS_PER_TILE, D), jnp.float32)
    h = x.astype(jnp.float32)
    for l in range(LAYERS):
        parts = _sc_agg(h, idx, zeros)
        h = _mlp(h, parts[0], parts[1], w1[l],
                 s1[l][None, :], t1[l][None, :],
                 w2[l], s2[l][None, :], t2[l][None, :],
                 l < LAYERS - 1)
    return h
